# baseline (speedup 1.0000x reference)
import functools

import jax
import jax.numpy as jnp
from jax import lax
from jax.experimental import pallas as pl
from jax.experimental.pallas import tpu as pltpu
from jax.experimental.pallas import tpu_sc as plsc

N = 10000
E = 320000
D = 128
NC, NS, L = 2, 16, 16
NW = NC * NS
N_PAD = 10240
RPT = N_PAD // NS
HC = 125
FCW = 2 * E // (NW * HC)
C = 64
EW = E // NW
EPW = 10240
CW = EPW // C
G = 16
NG = CW // G

_mesh = plsc.VectorSubcoreMesh(core_axis_name="c", subcore_axis_name="s")


@functools.partial(
    pl.kernel,
    out_type=jax.ShapeDtypeStruct((NC, N_PAD), jnp.float32),
    mesh=_mesh,
    scratch_types=[
        pltpu.VMEM((FCW, HC), jnp.int32),
        pltpu.VMEM((128,), jnp.float32),
        pltpu.VMEM((RPT,), jnp.float32),
        pltpu.VMEM_SHARED((N_PAD,), jnp.float32),
    ],
)
def _degree_kernel(idx_hbm, hist_hbm, idx_v, ones_v, stage_v, hist_sh):
    cid = lax.axis_index("c")
    sid = lax.axis_index("s")
    wid = cid * NS + sid
    for k in range(128 // L):
        ones_v[pl.ds(k * L, L)] = jnp.ones((L,), jnp.float32)
    for k in range(RPT // L):
        stage_v[pl.ds(k * L, L)] = jnp.zeros((L,), jnp.float32)
    pltpu.sync_copy(stage_v, hist_sh.at[pl.ds(sid * RPT, RPT)])
    plsc.subcore_barrier()
    pltpu.sync_copy(idx_hbm.at[wid], idx_v)

    @pl.loop(0, FCW)
    def _chunk(j):
        pltpu.sync_copy(ones_v.at[pl.ds(0, HC)], hist_sh.at[idx_v.at[j]],
                        add=True)

    plsc.subcore_barrier()
    pltpu.sync_copy(hist_sh.at[pl.ds(sid * RPT, RPT)], stage_v)
    pltpu.sync_copy(stage_v, hist_hbm.at[cid, pl.ds(sid * RPT, RPT)])


def _prescale_body(h_ref, x_ref, isd_ref, y_ref):
    deg = h_ref[0] + h_ref[1]
    isd = lax.rsqrt(jnp.maximum(deg, 1.0))[:N]
    isd_ref[...] = isd
    y_ref[:N] = x_ref[...] * isd
    y_ref[N:] = jnp.zeros((N_PAD - N, D), jnp.float32)


@functools.partial(
    pl.kernel,
    out_type=jax.ShapeDtypeStruct((NC, N_PAD, D), jnp.float32),
    mesh=_mesh,
    scratch_types=[
        pltpu.VMEM((G, C), jnp.int32),
        pltpu.VMEM((G, C), jnp.int32),
        [pltpu.VMEM((C, D), jnp.float32)] * 4,
        [pltpu.SemaphoreType.DMA] * 4,
        [pltpu.SemaphoreType.DMA] * 4,
        pltpu.VMEM_SHARED((N_PAD, D), jnp.float32),
    ],
)
def _spmm_kernel(y_hbm, ei_hbm, z_hbm, out_hbm,
                 e0g, e1g, rows, gs, ss, acc):
    cid = lax.axis_index("c")
    sid = lax.axis_index("s")
    wid = cid * NS + sid
    base = sid * RPT
    for j in range(RPT // 128):
        pltpu.sync_copy(z_hbm, acc.at[pl.ds(base + j * 128, 128)])
    plsc.subcore_barrier()

    def gref(i, c):
        return (e1g if i % 2 == 0 else e0g).at[c]

    def sref(i, c):
        return (e0g if i % 2 == 0 else e1g).at[c]

    def ring_body(m, last):
        sds = []
        for i in range(4):
            c = 2 * m + i // 2
            pltpu.make_async_copy(y_hbm.at[gref(i, c)], rows[i], gs[i]).wait()
            sds.append(pltpu.async_copy(rows[i], acc.at[sref(i, c)], ss[i],
                                        add=True))
        for i in range(4):
            sds[i].wait()
            if not last:
                cn = 2 * (m + 1) + i // 2
                pltpu.async_copy(y_hbm.at[gref(i, cn)], rows[i], gs[i])

    @pl.loop(0, NG)
    def _group(g):
        pltpu.sync_copy(ei_hbm.at[0, wid, pl.ds(g * G, G)], e0g)
        pltpu.sync_copy(ei_hbm.at[1, wid, pl.ds(g * G, G)], e1g)
        for i in range(4):
            pltpu.async_copy(y_hbm.at[gref(i, i // 2)], rows[i], gs[i])

        @pl.loop(0, G // 2 - 1)
        def _steady(m):
            ring_body(m, last=False)

        ring_body(G // 2 - 1, last=True)

    plsc.subcore_barrier()
    DR = 40
    rbuf = rows[0].at[pl.ds(0, DR)]
    for j in range(RPT // DR):
        pltpu.sync_copy(acc.at[pl.ds(base + j * DR, DR)], rbuf)
        pltpu.sync_copy(rbuf, out_hbm.at[cid, pl.ds(base + j * DR, DR)])


def _combine_body(p_ref, c_ref, o_ref):
    o_ref[...] = (p_ref[0, :N, :] + p_ref[1, :N, :]) * c_ref[...]


def kernel(x, edge_index):
    idx_flat = edge_index.reshape(NW, FCW, HC)

    ew = edge_index.reshape(2, NW, EW)
    pidx = jnp.arange(240, dtype=jnp.int32)[None, :]
    wrot = (jnp.arange(NW, dtype=jnp.int32) * 15)[:, None]
    pad = jnp.broadcast_to(N + (pidx + wrot) % (N_PAD - N), (2, NW, 240))
    ei = jnp.concatenate([ew, pad], axis=2).reshape(2, NW, CW, C)

    hist = _degree_kernel(idx_flat)

    isd_col, y = pl.pallas_call(
        _prescale_body,
        out_shape=(
            jax.ShapeDtypeStruct((N, 1), jnp.float32),
            jax.ShapeDtypeStruct((N_PAD, D), jnp.float32),
        ),
    )(hist.reshape(NC, N_PAD, 1), x)

    zrows = jnp.zeros((128, D), jnp.float32)
    part = _spmm_kernel(y, ei, zrows)

    out = pl.pallas_call(
        _combine_body,
        out_shape=jax.ShapeDtypeStruct((N, D), jnp.float32),
    )(part, isd_col)
    return out
